# Initial kernel scaffold; baseline (speedup 1.0000x reference)
import jax
import jax.numpy as jnp
from jax.experimental import pallas as pl


def kernel(x, edge_index, W1, b1, W2, b2):
    raise NotImplementedError("write your pallas kernel here")


# speedup vs baseline: 26.0474x; 26.0474x over previous
import functools

import jax
import jax.numpy as jnp
from jax import lax
from jax.experimental import pallas as pl
from jax.experimental.pallas import tpu as pltpu
from jax.experimental.pallas import tpu_sc as plsc

N = 10000
NPAD = 10240
E = 320000
D = 128
H = 64
C = 10
NCORE = 2
NSUB = 16
NW = NCORE * NSUB
CHUNK = 100
NCHUNK = E // (NW * CHUNK)
RPS = NPAD // NSUB

_MESH = plsc.VectorSubcoreMesh(
    core_axis_name="c", subcore_axis_name="s",
    num_cores=NCORE, num_subcores=NSUB)


@functools.partial(
    pl.kernel,
    out_type=jax.ShapeDtypeStruct((NCORE * NPAD, 16), jnp.float32),
    mesh=_MESH,
    scratch_types=[
        pltpu.VMEM((NCHUNK, CHUNK), jnp.int32),
        pltpu.VMEM((CHUNK, 16), jnp.float32),
        pltpu.VMEM((RPS, 16), jnp.float32),
        pltpu.VMEM_SHARED((NPAD, 16), jnp.float32),
    ],
    compiler_params=pltpu.CompilerParams(use_tc_tiling_on_sc=False),
)
def _sc_count(col_hbm, ones_hbm, zeros_hbm, out_hbm, colv, onesv, zbuf, acc):
    cid = lax.axis_index("c")
    sid = lax.axis_index("s")
    wid = cid * NSUB + sid
    pltpu.sync_copy(zeros_hbm, zbuf)
    pltpu.sync_copy(zbuf, acc.at[pl.ds(sid * RPS, RPS)])
    pltpu.sync_copy(ones_hbm, onesv)
    pltpu.sync_copy(col_hbm.at[wid], colv)
    plsc.subcore_barrier()

    def body(j, carry):
        pltpu.sync_copy(onesv, acc.at[colv.at[j]], add=True)
        return carry

    lax.fori_loop(0, NCHUNK, body, 0)
    plsc.subcore_barrier()
    pltpu.sync_copy(acc.at[pl.ds(sid * RPS, RPS)], zbuf)
    pltpu.sync_copy(zbuf, out_hbm.at[pl.ds(cid * NPAD + sid * RPS, RPS)])


def _make_sc_prop(F):
    @functools.partial(
        pl.kernel,
        out_type=jax.ShapeDtypeStruct((NCORE * NPAD, F), jnp.float32),
        mesh=_MESH,
        scratch_types=[
            pltpu.VMEM((NCHUNK, CHUNK), jnp.int32),
            pltpu.VMEM((NCHUNK, CHUNK), jnp.int32),
            pltpu.VMEM((CHUNK, F), jnp.float32),
            pltpu.VMEM((RPS, F), jnp.float32),
            pltpu.VMEM_SHARED((NPAD, F), jnp.float32),
            pltpu.SemaphoreType.DMA,
        ],
        compiler_params=pltpu.CompilerParams(use_tc_tiling_on_sc=False),
    )
    def prop(h_hbm, row_hbm, col_hbm, zeros_hbm, out_hbm,
             rowv, colv, buf, zbuf, acc, sem):
        cid = lax.axis_index("c")
        sid = lax.axis_index("s")
        wid = cid * NSUB + sid
        pltpu.sync_copy(zeros_hbm, zbuf)
        pltpu.sync_copy(zbuf, acc.at[pl.ds(sid * RPS, RPS)])
        pltpu.sync_copy(row_hbm.at[wid], rowv)
        pltpu.sync_copy(col_hbm.at[wid], colv)
        plsc.subcore_barrier()

        def body(j, carry):
            pltpu.async_copy(h_hbm.at[rowv.at[j]], buf, sem).wait()
            pltpu.sync_copy(buf, acc.at[colv.at[j]], add=True)
            return carry

        lax.fori_loop(0, NCHUNK, body, 0)
        plsc.subcore_barrier()
        pltpu.sync_copy(acc.at[pl.ds(sid * RPS, RPS)], zbuf)
        pltpu.sync_copy(zbuf, out_hbm.at[pl.ds(cid * NPAD + sid * RPS, RPS)])

    return prop


_sc_prop64 = _make_sc_prop(H)
_sc_prop16 = _make_sc_prop(16)


BLK = 512
GRID = NPAD // BLK


def _dinv_of(cnt_blk):
    deg = cnt_blk[0, :, 0] + cnt_blk[1, :, 0] + 1.0
    return lax.rsqrt(deg)


def _enc1_body(cnt_ref, x_ref, w1_ref, h1p_ref):
    dinv = _dinv_of(cnt_ref[...])
    h = jnp.dot(x_ref[...], w1_ref[...], preferred_element_type=jnp.float32)
    h1p_ref[...] = h * dinv[:, None]


def _enc2_body(cnt_ref, p1_ref, h1p_ref, b1_ref, w2_ref, h2p_ref):
    dinv = _dinv_of(cnt_ref[...])
    agg = p1_ref[0] + p1_ref[1] + h1p_ref[...]
    h1 = jnp.maximum(agg * dinv[:, None] + b1_ref[...], 0.0)
    h2 = jnp.dot(h1, w2_ref[...], preferred_element_type=jnp.float32)
    h2p_ref[...] = h2 * dinv[:, None]


def _out_body(cnt_ref, p2_ref, h2p_ref, b2_ref, out_ref):
    dinv = _dinv_of(cnt_ref[...])
    agg = p2_ref[0] + p2_ref[1] + h2p_ref[...]
    out_ref[...] = agg * dinv[:, None] + b2_ref[...]


def _cnt_spec():
    return pl.BlockSpec((2, BLK, 16), lambda i: (0, i, 0))


_tc_enc1 = pl.pallas_call(
    _enc1_body,
    grid=(GRID,),
    in_specs=[
        _cnt_spec(),
        pl.BlockSpec((BLK, D), lambda i: (i, 0)),
        pl.BlockSpec((D, H), lambda i: (0, 0)),
    ],
    out_specs=pl.BlockSpec((BLK, H), lambda i: (i, 0)),
    out_shape=jax.ShapeDtypeStruct((NPAD, H), jnp.float32),
)

_tc_enc2 = pl.pallas_call(
    _enc2_body,
    grid=(GRID,),
    in_specs=[
        _cnt_spec(),
        pl.BlockSpec((2, BLK, H), lambda i: (0, i, 0)),
        pl.BlockSpec((BLK, H), lambda i: (i, 0)),
        pl.BlockSpec((1, H), lambda i: (0, 0)),
        pl.BlockSpec((H, 16), lambda i: (0, 0)),
    ],
    out_specs=pl.BlockSpec((BLK, 16), lambda i: (i, 0)),
    out_shape=jax.ShapeDtypeStruct((NPAD, 16), jnp.float32),
)

_tc_out = pl.pallas_call(
    _out_body,
    grid=(GRID,),
    in_specs=[
        _cnt_spec(),
        pl.BlockSpec((2, BLK, 16), lambda i: (0, i, 0)),
        pl.BlockSpec((BLK, 16), lambda i: (i, 0)),
        pl.BlockSpec((1, 16), lambda i: (0, 0)),
    ],
    out_specs=pl.BlockSpec((BLK, 16), lambda i: (i, 0)),
    out_shape=jax.ShapeDtypeStruct((NPAD, 16), jnp.float32),
)


def kernel(x, edge_index, W1, b1, W2, b2):
    row3 = edge_index[0].reshape(NW, NCHUNK, CHUNK)
    col3 = edge_index[1].reshape(NW, NCHUNK, CHUNK)
    xpad = jnp.pad(x, ((0, NPAD - N), (0, 0)))
    ones16 = jnp.ones((CHUNK, 16), jnp.float32)
    z16 = jnp.zeros((RPS, 16), jnp.float32)
    z64 = jnp.zeros((RPS, H), jnp.float32)
    w2p = jnp.pad(W2, ((0, 0), (0, 16 - C)))
    b2p = jnp.pad(b2, (0, 16 - C)).reshape(1, 16)

    cnt = _sc_count(col3, ones16, z16).reshape(NCORE, NPAD, 16)
    h1p = _tc_enc1(cnt, xpad, W1)
    p1 = _sc_prop64(h1p, row3, col3, z64).reshape(NCORE, NPAD, H)
    h2p = _tc_enc2(cnt, p1, h1p, b1.reshape(1, H), w2p)
    p2 = _sc_prop16(h2p, row3, col3, z16).reshape(NCORE, NPAD, 16)
    out = _tc_out(cnt, p2, h2p, b2p)
    return out[:N, :C]
